# Initial kernel scaffold; baseline (speedup 1.0000x reference)
import jax
import jax.numpy as jnp
from jax.experimental import pallas as pl


def kernel(x, connection_indices, kernel, bias):
    raise NotImplementedError("write your pallas kernel here")


# speedup vs baseline: 27.0450x; 27.0450x over previous
import functools

import jax
import jax.numpy as jnp
from jax import lax
from jax.experimental import pallas as pl
from jax.experimental.pallas import tpu as pltpu
from jax.experimental.pallas import tpu_sc as plsc

B = 8
N_IN = 196608
N_OUT = 49152
K = 4
C_IN = 16
FILTERS = 32
NK = N_OUT * K
D = B * C_IN

NC = 2
NS = 16
NW = NC * NS
PER_W = NK // NW
IDX_MINOR = 128
IDX_ROWS = PER_W // IDX_MINOR
CHUNK_ROWS = 2
CHUNK = CHUNK_ROWS * IDX_MINOR
NCHUNK = IDX_ROWS // CHUNK_ROWS

_sc_mesh = plsc.VectorSubcoreMesh(core_axis_name="c", subcore_axis_name="s")


@functools.partial(
    pl.kernel,
    out_type=jax.ShapeDtypeStruct((NK, D), jnp.float32),
    mesh=_sc_mesh,
    scratch_types=[
        pltpu.VMEM((IDX_ROWS, IDX_MINOR), jnp.int32),
        pltpu.VMEM((CHUNK, D), jnp.float32),
        pltpu.SemaphoreType.DMA,
    ],
)
def _sc_gather(xt_hbm, idx_hbm, z_hbm, idx_v, rows_v, sem):
    wid = lax.axis_index("s") * NC + lax.axis_index("c")
    pltpu.sync_copy(idx_hbm.at[wid], idx_v)

    def step(ch, carry):
        for j in range(CHUNK_ROWS):
            pltpu.async_copy(
                xt_hbm.at[idx_v.at[ch * CHUNK_ROWS + j]],
                rows_v.at[pl.ds(j * IDX_MINOR, IDX_MINOR)],
                sem,
            )
        pltpu.make_async_copy(z_hbm.at[pl.ds(0, CHUNK)], rows_v, sem).wait()
        pltpu.sync_copy(rows_v, z_hbm.at[pl.ds(wid * PER_W + ch * CHUNK, CHUNK)])
        return carry

    lax.fori_loop(0, NCHUNK, step, 0)


TILE = 2048
KW = K * D
NF = B * FILTERS


def _mm_body(z_ref, w_ref, bias_ref, o_ref):
    o_ref[...] = (
        jnp.dot(z_ref[...], w_ref[...], preferred_element_type=jnp.float32)
        + bias_ref[...]
    )


_matmul = pl.pallas_call(
    _mm_body,
    grid=(N_OUT // TILE,),
    in_specs=[
        pl.BlockSpec((TILE, KW), lambda i: (i, 0)),
        pl.BlockSpec((KW, NF), lambda i: (0, 0)),
        pl.BlockSpec((1, NF), lambda i: (0, 0)),
    ],
    out_specs=pl.BlockSpec((TILE, NF), lambda i: (i, 0)),
    out_shape=jax.ShapeDtypeStruct((N_OUT, NF), jnp.float32),
)


def kernel(x, connection_indices, kernel, bias):
    xt = x.transpose(1, 0, 2).reshape(N_IN, D)
    idx = connection_indices.astype(jnp.int32).reshape(NW, IDX_ROWS, IDX_MINOR)
    z = _sc_gather(xt, idx)
    w_big = (
        jnp.eye(B, dtype=jnp.float32)[None, :, None, :, None]
        * kernel[:, None, :, None, :]
    ).reshape(KW, NF)
    bias_t = jnp.tile(bias, B).reshape(1, NF)
    y = _matmul(z.reshape(N_OUT, KW), w_big, bias_t)
    return y.reshape(N_OUT, B, FILTERS).transpose(1, 0, 2)
